# Initial kernel scaffold; baseline (speedup 1.0000x reference)
import jax
import jax.numpy as jnp
from jax.experimental import pallas as pl


def kernel(node_x, edge_index, edge_bank, W_node, b_node, W_edge, b_edge, g_ca, b_ca_ln, W_ca, b_ca, W_tok, b_tok, W_r1, b_r1, g_r, b_r_ln, W_r2, b_r2, head_prompts):
    raise NotImplementedError("write your pallas kernel here")


# speedup vs baseline: 3.8573x; 3.8573x over previous
import functools

import jax
import jax.numpy as jnp
from jax import lax
from jax.experimental import pallas as pl
from jax.experimental.pallas import tpu as pltpu
from jax.experimental.pallas import tpu_sc as plsc

F32 = jnp.float32
_TAU = 2.0
_BUDGET = 0.12
_EPS = 1e-5


def _gelu(x):
    return 0.5 * x * (1.0 + lax.erf(x * (2.0 ** -0.5)))


def _node_body(x_ref, wn_ref, bn_ref, wts_ref, wtd_ref, o_ref):
    nf = jnp.dot(x_ref[0], wn_ref[...], preferred_element_type=F32)
    nf = nf + bn_ref[0:1, :]
    o_ref[0, 0] = jnp.dot(nf, wts_ref[...], preferred_element_type=F32)
    o_ref[1, 0] = jnp.dot(nf, wtd_ref[...], preferred_element_type=F32)


def _node_proj(node_x, w_node, bn2, wt_s, wt_d):
    b, r, hid = node_x.shape
    dm = w_node.shape[1]
    tr = 256
    return pl.pallas_call(
        _node_body,
        grid=(b, r // tr),
        in_specs=[
            pl.BlockSpec((1, tr, hid), lambda bb, i: (bb, i, 0)),
            pl.BlockSpec((hid, dm), lambda bb, i: (0, 0)),
            pl.BlockSpec((8, dm), lambda bb, i: (0, 0)),
            pl.BlockSpec((dm, dm), lambda bb, i: (0, 0)),
            pl.BlockSpec((dm, dm), lambda bb, i: (0, 0)),
        ],
        out_specs=pl.BlockSpec((2, 1, tr, dm), lambda bb, i: (0, bb, i, 0)),
        out_shape=jax.ShapeDtypeStruct((2, b, r, dm), F32),
    )(node_x, w_node, bn2, wt_s, wt_d)


def _sc_gather(idx2d, p2, *, BR, BE, E, R, DM):
    info = plsc.get_sparse_core_info()
    nc = info.num_cores
    nw = nc * info.num_subcores
    chunk = (2 * BE) // nw
    nch = 64
    nsub = chunk // nch
    mesh = plsc.VectorSubcoreMesh(core_axis_name="c", subcore_axis_name="s")

    @functools.partial(
        pl.kernel,
        mesh=mesh,
        out_type=jax.ShapeDtypeStruct((2 * BE, DM), F32),
        scratch_types=[
            pltpu.VMEM((nsub, nch), jnp.int32),
            pltpu.VMEM((nch, DM), F32),
            pltpu.SemaphoreType.DMA,
        ],
    )
    def gather_k(idx_hbm, p2_hbm, g2_hbm, idxv, buf, sem):
        cc = lax.axis_index("c")
        ss = lax.axis_index("s")
        wid = ss * nc + cc
        rowbase = wid * nsub
        pltpu.sync_copy(idx_hbm.at[pl.ds(rowbase, nsub)], idxv)
        base_e = wid * chunk
        off = (base_e // BE) * BR + ((base_e % BE) // E) * R
        for ci in range(nsub):
            for i in range(nch // 16):
                sl = pl.ds(i * 16, 16)
                idxv[ci, sl] = idxv[ci, sl] + off
        for ci in range(nsub):
            cp = pltpu.async_copy(p2_hbm.at[idxv.at[ci]], buf, sem)
            cp.wait()
            pltpu.sync_copy(buf, g2_hbm.at[pl.ds(base_e + ci * nch, nch)])

    return gather_k(idx2d, p2)


def _edge_body(eb_ref, gs_ref, gd_ref, wea_ref, wca_ref, wte_ref,
               wr1_ref, wr2_ref, hpt_ref, vdm_ref,
               probs_ref, rid_ref, score_ref, *, nexp):
    c32 = eb_ref[...]
    b_edge = vdm_ref[0:1, :]
    g_ca = vdm_ref[1:2, :]
    b_ca_ln = vdm_ref[2:3, :]
    b_ca = vdm_ref[3:4, :]
    b_tok = vdm_ref[4:5, :]
    b_r1 = vdm_ref[5:6, :]
    g_r = vdm_ref[6:7, :]
    b_r_ln = vdm_ref[7:8, :]
    b_r2 = vdm_ref[8:9, :]

    ef = jnp.dot(c32, wea_ref[...], preferred_element_type=F32) + b_edge
    mu = jnp.mean(ef, axis=-1, keepdims=True)
    va = jnp.mean((ef - mu) ** 2, axis=-1, keepdims=True)
    ln = (ef - mu) / jnp.sqrt(va + _EPS) * g_ca + b_ca_ln
    ef = ef + _gelu(jnp.dot(ln, wca_ref[...], preferred_element_type=F32)
                    + b_ca)
    token = (jnp.dot(ef, wte_ref[...], preferred_element_type=F32)
             + gs_ref[...] + gd_ref[...] + b_tok)
    h = _gelu(jnp.dot(token, wr1_ref[...], preferred_element_type=F32) + b_r1)
    mu = jnp.mean(h, axis=-1, keepdims=True)
    va = jnp.mean((h - mu) ** 2, axis=-1, keepdims=True)
    h = (h - mu) / jnp.sqrt(va + _EPS) * g_r + b_r_ln
    routed = _gelu(jnp.dot(h, wr2_ref[...], preferred_element_type=F32)
                   + b_r2)
    logits = jnp.dot(routed, hpt_ref[...], preferred_element_type=F32)
    col = lax.broadcasted_iota(jnp.int32, logits.shape, 1)
    logits = jnp.where(col < nexp, logits, -jnp.inf)
    x = logits / _TAU
    m = jnp.max(x, axis=-1, keepdims=True)
    ex = jnp.exp(x - m)
    p = ex / jnp.sum(ex, axis=-1, keepdims=True)
    probs_ref[...] = p
    maxv = jnp.max(logits, axis=-1, keepdims=True)
    ridv = jnp.min(jnp.where(logits == maxv, col, nexp), axis=-1)
    rid_ref[...] = ridv.reshape(rid_ref.shape)
    score_ref[...] = jnp.max(p, axis=-1).reshape(score_ref.shape)


def _edge_route(eb_aug, g2, wea, w_ca, wt_e, w_r1, w_r2, hpt, vdm, *, nexp):
    be, _ = eb_aug.shape
    dm = w_ca.shape[0]
    te = 512
    nb = be // te
    kfn = functools.partial(_edge_body, nexp=nexp)
    return pl.pallas_call(
        kfn,
        grid=(nb,),
        in_specs=[
            pl.BlockSpec((te, 32), lambda i: (i, 0)),
            pl.BlockSpec((te, dm), lambda i: (i, 0)),
            pl.BlockSpec((te, dm), lambda i: (nb + i, 0)),
            pl.BlockSpec((32, dm), lambda i: (0, 0)),
            pl.BlockSpec((dm, dm), lambda i: (0, 0)),
            pl.BlockSpec((dm, dm), lambda i: (0, 0)),
            pl.BlockSpec((dm, dm), lambda i: (0, 0)),
            pl.BlockSpec((dm, dm), lambda i: (0, 0)),
            pl.BlockSpec((dm, 16), lambda i: (0, 0)),
            pl.BlockSpec((16, dm), lambda i: (0, 0)),
        ],
        out_specs=[
            pl.BlockSpec((te, 16), lambda i: (i, 0)),
            pl.BlockSpec((1, 1, te), lambda i: (i, 0, 0)),
            pl.BlockSpec((1, 1, te), lambda i: (i, 0, 0)),
        ],
        out_shape=[
            jax.ShapeDtypeStruct((be, 16), F32),
            jax.ShapeDtypeStruct((nb, 1, te), jnp.int32),
            jax.ShapeDtypeStruct((nb, 1, te), F32),
        ],
    )(eb_aug, g2, g2, wea, w_ca, wt_e, w_r1, w_r2, hpt, vdm)


def _mask_body(score_ref, rid_ref, ei_ref, typed_ref, keep_ref, *, k, prune):
    s = score_ref[...]
    bits = lax.bitcast_convert_type(s, jnp.int32)
    b = s.shape[0]

    def body(_, lohi):
        lo, hi = lohi
        mid = lo + (hi - lo + 1) // 2
        cnt = jnp.sum((bits >= mid).astype(jnp.int32), axis=-1, keepdims=True)
        ok = cnt >= k
        return jnp.where(ok, mid, lo), jnp.where(ok, hi, mid - 1)

    lo0 = jnp.zeros((b, 1), jnp.int32)
    hi0 = jnp.full((b, 1), 0x7F800000, jnp.int32)
    lo, _ = lax.fori_loop(0, 31, body, (lo0, hi0))
    keep = (bits >= lo) & (rid_ref[...] != prune)
    keep_ref[...] = keep.astype(jnp.int32)
    typed_ref[...] = ei_ref[...] * keep[:, None, :].astype(jnp.int32)


def _mask(score, rid, edge_index, *, k, prune):
    b, e = score.shape
    kfn = functools.partial(_mask_body, k=k, prune=prune)
    return pl.pallas_call(
        kfn,
        out_shape=[
            jax.ShapeDtypeStruct((b, 2, e), jnp.int32),
            jax.ShapeDtypeStruct((b, e), jnp.int32),
        ],
    )(score, rid, edge_index)


def kernel(node_x, edge_index, edge_bank, W_node, b_node, W_edge, b_edge,
           g_ca, b_ca_ln, W_ca, b_ca, W_tok, b_tok,
           W_r1, b_r1, g_r, b_r_ln, W_r2, b_r2, head_prompts):
    B, R, HID = node_x.shape
    E = edge_index.shape[2]
    DM = W_node.shape[1]
    M = W_edge.shape[0]
    NEXP = head_prompts.shape[0]
    BE = B * E
    BR = B * R
    k = max(1, int(E * _BUDGET))
    prune = NEXP - 1

    wt_e = W_tok[:DM]
    wt_s = W_tok[DM:2 * DM]
    wt_d = W_tok[2 * DM:3 * DM]
    bn2 = jnp.zeros((8, DM), F32).at[0].set(b_node)

    wea = jnp.zeros((32, DM), F32).at[:M].set(W_edge)
    vdm = jnp.zeros((16, DM), F32)
    for i, v in enumerate([b_edge, g_ca, b_ca_ln, b_ca, b_tok, b_r1,
                           g_r, b_r_ln, b_r2]):
        vdm = vdm.at[i].set(v)
    hpt = jnp.zeros((DM, 16), F32).at[:, :NEXP].set(head_prompts.T)
    eb_aug = jnp.concatenate(
        [edge_bank.reshape(BE, M), jnp.zeros((BE, 32 - M), F32)], axis=1)

    p2 = _node_proj(node_x, W_node, bn2, wt_s, wt_d).reshape(2 * BR, DM)
    idx2d = edge_index.transpose(1, 0, 2).reshape((2 * BE) // 64, 64)
    g2 = _sc_gather(idx2d, p2, BR=BR, BE=BE, E=E, R=R, DM=DM)

    probs16, rid3, score3 = _edge_route(
        eb_aug, g2, wea, W_ca, wt_e, W_r1, W_r2, hpt, vdm, nexp=NEXP)

    rid = rid3.reshape(B, E)
    score = score3.reshape(B, E)
    typed, keepi = _mask(score, rid, edge_index, k=k, prune=prune)

    route_probs = probs16.reshape(B, E, 16)[..., :NEXP]
    return typed, rid, keepi.astype(bool), route_probs
